# baseline (speedup 1.0000x reference)
import functools

import numpy as np
import jax
import jax.numpy as jnp
from jax import lax
from jax.experimental import pallas as pl
from jax.experimental.pallas import tpu as pltpu
from jax.experimental.pallas import tpu_sc as plsc

VOCAB = 1000000
D = 64
B = 4096
L = 200
MAX_LEN = 512

NC = 2
NS = 16
NW = NC * NS

ROWS_PER_W = B // NW
CHUNK_ROWS = 2
CH = CHUNK_ROWS * L
NCHUNK = ROWS_PER_W // CHUNK_ROWS
GATHER_PIECES = [(o, min(128, CH - o)) for o in range(0, CH, 128)]


def _pos_encoding(max_len, d):
    pos = np.arange(max_len, dtype=np.float32)[:, None]
    div = np.exp(np.arange(0, d, 2, dtype=np.float32) * (-np.log(10000.0) / d))
    pe = np.zeros((max_len, d), dtype=np.float32)
    pe[:, 0::2] = np.sin(pos * div)
    pe[:, 1::2] = np.cos(pos * div)
    return pe


_PE = jnp.asarray(_pos_encoding(MAX_LEN, D)[:L])
_SCALE = float(np.sqrt(np.float32(D)))


@functools.partial(
    pl.kernel,
    out_type=jax.ShapeDtypeStruct((B * L, D), jnp.float32),
    mesh=plsc.VectorSubcoreMesh(
        core_axis_name="c", subcore_axis_name="s", num_cores=NC, num_subcores=NS
    ),
    scratch_types=[
        pltpu.VMEM((CH,), jnp.int32),
        pltpu.VMEM((CH, D), jnp.float32),
        pltpu.VMEM((L, D), jnp.float32),
        pltpu.SemaphoreType.DMA,
    ],
    compiler_params=pltpu.CompilerParams(use_tc_tiling_on_sc=False),
)
def _emb_kernel(seq_hbm, w_hbm, pe_hbm, out_hbm, idx_v, rows_v, pe_v, sem):
    wid = lax.axis_index("s") * NC + lax.axis_index("c")
    pltpu.sync_copy(pe_hbm, pe_v)

    def chunk_body(g, carry):
        base = wid * (ROWS_PER_W * L) + g * CH
        pltpu.sync_copy(seq_hbm.at[pl.ds(base, CH)], idx_v)
        cps = [
            pltpu.async_copy(
                w_hbm.at[idx_v.at[pl.ds(o, n)]], rows_v.at[pl.ds(o, n)], sem
            )
            for o, n in GATHER_PIECES
        ]
        for cp in cps:
            cp.wait()

        def pos_body(l, c):
            for r in range(CHUNK_ROWS):
                row = r * L + l
                for j in range(D // 16):
                    sl = pl.ds(j * 16, 16)
                    rows_v[row, sl] = rows_v[row, sl] * _SCALE + pe_v[l, sl]
            return c

        lax.fori_loop(0, L, pos_body, 0)
        pltpu.sync_copy(rows_v, out_hbm.at[pl.ds(base, CH)])
        return carry

    lax.fori_loop(0, NCHUNK, chunk_body, 0)


def kernel(seq, weight):
    out = _emb_kernel(seq.reshape(B * L), weight, _PE)
    return out.reshape(B, L, D)


# speedup vs baseline: 1.1176x; 1.1176x over previous
import functools

import numpy as np
import jax
import jax.numpy as jnp
from jax import lax
from jax.experimental import pallas as pl
from jax.experimental.pallas import tpu as pltpu
from jax.experimental.pallas import tpu_sc as plsc

VOCAB = 1000000
D = 64
B = 4096
L = 200
MAX_LEN = 512

NC = 2
NS = 16
NW = NC * NS

ROWS_PER_W = B // NW
CHUNK_ROWS = 2
CH = CHUNK_ROWS * L
NCHUNK = ROWS_PER_W // CHUNK_ROWS
NBUF = 4
GATHER_PIECES = [(o, min(128, CH - o)) for o in range(0, CH, 128)]


def _pos_encoding(max_len, d):
    pos = np.arange(max_len, dtype=np.float32)[:, None]
    div = np.exp(np.arange(0, d, 2, dtype=np.float32) * (-np.log(10000.0) / d))
    pe = np.zeros((max_len, d), dtype=np.float32)
    pe[:, 0::2] = np.sin(pos * div)
    pe[:, 1::2] = np.cos(pos * div)
    return pe


_PE = jnp.asarray(_pos_encoding(MAX_LEN, D)[:L])
_SCALE = float(np.sqrt(np.float32(D)))


@functools.partial(
    pl.kernel,
    out_type=jax.ShapeDtypeStruct((B * L, D), jnp.float32),
    mesh=plsc.VectorSubcoreMesh(
        core_axis_name="c", subcore_axis_name="s", num_cores=NC, num_subcores=NS
    ),
    scratch_types=[
        [pltpu.VMEM((CH,), jnp.int32) for _ in range(NBUF)],
        [pltpu.VMEM((CH, D), jnp.float32) for _ in range(NBUF)],
        pltpu.VMEM((L, D), jnp.float32),
        [pltpu.SemaphoreType.DMA for _ in range(NBUF)],
        [pltpu.SemaphoreType.DMA for _ in range(NBUF)],
    ],
    compiler_params=pltpu.CompilerParams(use_tc_tiling_on_sc=False),
)
def _emb_kernel(seq_hbm, w_hbm, pe_hbm, out_hbm,
                idx_bufs, rows_bufs, pe_v, gsems, osems):
    wid = lax.axis_index("s") * NC + lax.axis_index("c")
    wbase = wid * (ROWS_PER_W * L)
    pltpu.sync_copy(pe_hbm, pe_v)

    def fire_gather(g, p):
        base = wbase + g * CH
        pltpu.sync_copy(seq_hbm.at[pl.ds(base, CH)], idx_bufs[p])
        for o, n in GATHER_PIECES:
            pltpu.async_copy(
                w_hbm.at[idx_bufs[p].at[pl.ds(o, n)]],
                rows_bufs[p].at[pl.ds(o, n)],
                gsems[p],
            )

    def wait_gather(p):
        pltpu.make_async_copy(
            w_hbm.at[pl.ds(0, CH)], rows_bufs[p], gsems[p]
        ).wait()

    def wait_out(p):
        pltpu.make_async_copy(
            rows_bufs[p], out_hbm.at[pl.ds(0, CH)], osems[p]
        ).wait()

    fire_gather(0, 0)
    fire_gather(1, 1)

    def outer(h, carry):
        for p in range(NBUF):
            g = h * NBUF + p
            p2 = (p + 2) % NBUF

            @pl.when(g + 2 < NCHUNK)
            def _():
                @pl.when(g >= 2)
                def _():
                    wait_out(p2)
                fire_gather(g + 2, p2)

            wait_gather(p)

            def pos_body(l, c):
                pes = [pe_v[l, pl.ds(j * 16, 16)] for j in range(D // 16)]
                for r in range(CHUNK_ROWS):
                    row = r * L + l
                    for j in range(D // 16):
                        sl = pl.ds(j * 16, 16)
                        rows_bufs[p][row, sl] = (
                            rows_bufs[p][row, sl] * _SCALE + pes[j]
                        )
                return c

            lax.fori_loop(0, L, pos_body, 0, unroll=2)

            pltpu.async_copy(
                rows_bufs[p], out_hbm.at[pl.ds(wbase + g * CH, CH)], osems[p]
            )
        return carry

    lax.fori_loop(0, NCHUNK // NBUF, outer, 0)

    for p in range(NBUF):
        wait_out(p)


def kernel(seq, weight):
    out = _emb_kernel(seq.reshape(B * L), weight, _PE)
    return out.reshape(B, L, D)
